# Initial kernel scaffold; baseline (speedup 1.0000x reference)
import jax
import jax.numpy as jnp
from jax.experimental import pallas as pl


def kernel(boxes, scores):
    raise NotImplementedError("write your pallas kernel here")


# speedup vs baseline: 19.5165x; 19.5165x over previous
import jax
import jax.numpy as jnp
from jax.experimental import pallas as pl
from jax.experimental.pallas import tpu as pltpu

N = 20000
MAX_OUT = 100
IOU_THR = 0.5
SCORE_THR = 0.05

ROWS = 160
COLS = 128
PADN = ROWS * COLS


def _nms_body(x1_ref, y1_ref, x2_ref, y2_ref, s_ref,
              ox1_ref, oy1_ref, ox2_ref, oy2_ref, osc_ref, oidx_ref):
    x1 = x1_ref[...]
    y1 = y1_ref[...]
    x2 = x2_ref[...]
    y2 = y2_ref[...]
    s0 = s_ref[...]
    s_init = jnp.where(s0 >= SCORE_THR, s0, -1.0)
    area = (x2 - x1) * (y2 - y1)
    ridx = jax.lax.broadcasted_iota(jnp.int32, (ROWS, COLS), 0)
    cidx = jax.lax.broadcasted_iota(jnp.int32, (ROWS, COLS), 1)
    idxf = (ridx * COLS + cidx).astype(jnp.float32)
    big = jnp.float32(3.0e7)

    def body(i, s):
        m = jnp.max(s)
        amin = jnp.min(jnp.where(s == m, idxf, big))
        onehot = (idxf == amin).astype(jnp.float32)
        bx1 = jnp.sum(x1 * onehot)
        by1 = jnp.sum(y1 * onehot)
        bx2 = jnp.sum(x2 * onehot)
        by2 = jnp.sum(y2 * onehot)
        barea = (bx2 - bx1) * (by2 - by1)
        xx1 = jnp.maximum(x1, bx1)
        yy1 = jnp.maximum(y1, by1)
        xx2 = jnp.minimum(x2, bx2)
        yy2 = jnp.minimum(y2, by2)
        inter = jnp.maximum(xx2 - xx1, 0.0) * jnp.maximum(yy2 - yy1, 0.0)
        iou = inter / (area + barea - inter + 1e-8)
        valid = m > 0.0
        sup = jnp.logical_and(valid, iou >= IOU_THR)
        s = jnp.where(jnp.logical_or(sup, idxf == amin), -1.0, s)
        vf = jnp.where(valid, jnp.float32(1.0), jnp.float32(0.0))
        ox1_ref[pl.ds(i, 1), :] = (vf * bx1).reshape(1, 1)
        oy1_ref[pl.ds(i, 1), :] = (vf * by1).reshape(1, 1)
        ox2_ref[pl.ds(i, 1), :] = (vf * bx2).reshape(1, 1)
        oy2_ref[pl.ds(i, 1), :] = (vf * by2).reshape(1, 1)
        osc_ref[pl.ds(i, 1), :] = (vf * m).reshape(1, 1)
        oidx_ref[pl.ds(i, 1), :] = jnp.where(valid, amin, jnp.float32(-1.0)).reshape(1, 1)
        return s

    jax.lax.fori_loop(0, MAX_OUT, body, s_init)


def kernel(boxes, scores):
    pad = PADN - N
    x1 = jnp.pad(boxes[:, 0], (0, pad)).reshape(ROWS, COLS)
    y1 = jnp.pad(boxes[:, 1], (0, pad)).reshape(ROWS, COLS)
    x2 = jnp.pad(boxes[:, 2], (0, pad)).reshape(ROWS, COLS)
    y2 = jnp.pad(boxes[:, 3], (0, pad)).reshape(ROWS, COLS)
    s = jnp.pad(scores, (0, pad), constant_values=-1.0).reshape(ROWS, COLS)

    outs = pl.pallas_call(
        _nms_body,
        out_shape=[jax.ShapeDtypeStruct((MAX_OUT, 1), jnp.float32)] * 6,
    )(x1, y1, x2, y2, s)
    ox1, oy1, ox2, oy2, osc, oidx = outs
    kept_boxes = jnp.concatenate([ox1, oy1, ox2, oy2], axis=1)
    kept_scores = osc[:, 0]
    kept_idx = oidx[:, 0].astype(jnp.int32)
    return kept_boxes, kept_scores, kept_idx
